# baseline (speedup 1.0000x reference)
import functools

import jax
import jax.numpy as jnp
from jax.experimental import pallas as pl
from jax.experimental.pallas import tpu as pltpu

MARGIN = 10.0


def _loss_kernel(labels_ref, emb_ref, cen_ref, out_ref, acc_ref, *, n_blocks, k):
    i = pl.program_id(0)

    @pl.when(i == 0)
    def _init():
        acc_ref[...] = jnp.zeros_like(acc_ref)

    e = emb_ref[...]
    c = cen_ref[...]
    labels = labels_ref[...]

    ab = jax.lax.dot_general(
        e, c, (((1,), (1,)), ((), ())),
        preferred_element_type=jnp.float32,
        precision=jax.lax.Precision.HIGHEST,
    )
    aa = jnp.sum(e * e, axis=1, keepdims=True)
    bb = jnp.sum(c * c, axis=1)[None, :]
    d2 = jnp.maximum(aa + bb - 2.0 * ab, 1e-12)
    d = jnp.sqrt(d2)
    r = (MARGIN - d) ** 2

    onehot = (labels == jax.lax.broadcasted_iota(jnp.int32, (1, k), 1)
              ).astype(jnp.float32)
    row_tot = jnp.sum(r, axis=1, keepdims=True)

    acc_ref[0, :] += jnp.sum(onehot, axis=0)
    acc_ref[1, :] += jnp.sum(onehot * d2, axis=0)
    acc_ref[2, :] += jnp.sum(onehot * r, axis=0)
    acc_ref[3, :] += jnp.sum(onehot * row_tot, axis=0)

    @pl.when(i == n_blocks - 1)
    def _finish():
        counts = acc_ref[0, :]
        attr = acc_ref[1, :] / jnp.maximum(counts, 1.0)
        rep = (acc_ref[3, :] - acc_ref[2, :]) / jnp.maximum(counts * (k - 1), 1.0)
        valid = counts > 0.0
        n_valid = jnp.sum(valid.astype(jnp.float32))
        total = (jnp.sum(jnp.where(valid, attr, 0.0))
                 + jnp.sum(jnp.where(valid, rep, 0.0))) / n_valid
        out_ref[...] = total[None, None]


def kernel(embeddings, cluster_labels, centroids):
    n, d_feat = embeddings.shape
    k = centroids.shape[0]
    block = 4000
    n_blocks = n // block
    assert n_blocks * block == n

    labels2 = jnp.asarray(cluster_labels, jnp.int32).reshape(n, 1)

    out = pl.pallas_call(
        functools.partial(_loss_kernel, n_blocks=n_blocks, k=k),
        grid=(n_blocks,),
        in_specs=[
            pl.BlockSpec((block, 1), lambda i: (i, 0)),
            pl.BlockSpec((block, d_feat), lambda i: (i, 0)),
            pl.BlockSpec((k, d_feat), lambda i: (0, 0)),
        ],
        out_specs=pl.BlockSpec((1, 1), lambda i: (0, 0)),
        out_shape=jax.ShapeDtypeStruct((1, 1), jnp.float32),
        scratch_shapes=[pltpu.VMEM((8, k), jnp.float32)],
    )(labels2, embeddings, centroids)
    return out[0, 0]
